# speedup vs baseline: 8.4153x; 8.4153x over previous
import functools

import jax
import jax.numpy as jnp
from jax import lax
from jax.experimental import pallas as pl
from jax.experimental.pallas import tpu as pltpu
from jax.experimental.pallas import tpu_sc as plsc

N = 10000
E = 320000
D = 128
NP = 10240
B = 128
NB = E // B
NC = 2
NS = 16
NW = NC * NS
ROWS_PER_TILE = NP // NS

_mesh = plsc.VectorSubcoreMesh(core_axis_name="c", subcore_axis_name="s")


def _worker_batches(wid):
    full, rem = divmod(NB, NW)
    return full + jnp.where(wid < rem, 1, 0).astype(jnp.int32)


@functools.partial(
    pl.kernel,
    mesh=_mesh,
    out_type=jax.ShapeDtypeStruct((NC, 2, NP), jnp.float32),
    scratch_types=[
        pltpu.VMEM((B,), jnp.int32),
        pltpu.VMEM((B,), jnp.float32),
        pltpu.VMEM((ROWS_PER_TILE,), jnp.float32),
        pltpu.VMEM_SHARED((NP,), jnp.float32),
        pltpu.VMEM_SHARED((NP,), jnp.float32),
    ],
)
def _degree_kernel(edge_hbm, out_hbm, idx_v, ones_v, zb_v, dsrc_sh, ddst_sh):
    c = lax.axis_index("c")
    s = lax.axis_index("s")
    wid = s * NC + c

    one16 = jnp.ones((16,), jnp.float32)
    zero16 = jnp.zeros((16,), jnp.float32)
    for k in range(B // 16):
        ones_v[pl.ds(k * 16, 16)] = one16

    def zfill(j, carry):
        zb_v[pl.ds(j * 16, 16)] = zero16
        return carry

    lax.fori_loop(0, ROWS_PER_TILE // 16, zfill, None)
    pltpu.sync_copy(zb_v, dsrc_sh.at[pl.ds(s * ROWS_PER_TILE, ROWS_PER_TILE)])
    pltpu.sync_copy(zb_v, ddst_sh.at[pl.ds(s * ROWS_PER_TILE, ROWS_PER_TILE)])
    plsc.subcore_barrier()

    nb = _worker_batches(wid)

    def body(i, carry):
        off = (wid + i * NW) * B
        pltpu.sync_copy(edge_hbm.at[0, pl.ds(off, B)], idx_v)
        pltpu.sync_copy(ones_v, dsrc_sh.at[idx_v], add=True)
        pltpu.sync_copy(edge_hbm.at[1, pl.ds(off, B)], idx_v)
        pltpu.sync_copy(ones_v, ddst_sh.at[idx_v], add=True)
        return carry

    lax.fori_loop(0, nb, body, None)
    plsc.subcore_barrier()

    r0 = s * ROWS_PER_TILE
    pltpu.sync_copy(dsrc_sh.at[pl.ds(r0, ROWS_PER_TILE)],
                    out_hbm.at[c, 0, pl.ds(r0, ROWS_PER_TILE)])
    pltpu.sync_copy(ddst_sh.at[pl.ds(r0, ROWS_PER_TILE)],
                    out_hbm.at[c, 1, pl.ds(r0, ROWS_PER_TILE)])


@functools.partial(
    pl.kernel,
    mesh=_mesh,
    out_type=jax.ShapeDtypeStruct((NC, NP, D), jnp.float32),
    scratch_types=[
        pltpu.VMEM((B,), jnp.int32),
        pltpu.VMEM((B,), jnp.int32),
        pltpu.VMEM((B, D), jnp.float32),
        pltpu.VMEM_SHARED((NP, D), jnp.float32),
        pltpu.SemaphoreType.DMA,
    ],
)
def _aggregate_kernel(h_hbm, edge_hbm, out_hbm, sidx_v, didx_v, rows_v,
                      acc_sh, sem):
    c = lax.axis_index("c")
    s = lax.axis_index("s")
    wid = s * NC + c

    zero16 = jnp.zeros((16,), jnp.float32)

    def zfill(j, carry):
        r = j // (D // 16)
        k = (j % (D // 16)) * 16
        rows_v[r, pl.ds(k, 16)] = zero16
        return carry

    lax.fori_loop(0, B * (D // 16), zfill, None)
    r0 = s * ROWS_PER_TILE
    for j in range(ROWS_PER_TILE // B):
        pltpu.sync_copy(rows_v, acc_sh.at[pl.ds(r0 + j * B, B)])
    plsc.subcore_barrier()

    nb = _worker_batches(wid)

    def body(i, carry):
        off = (wid + i * NW) * B
        pltpu.sync_copy(edge_hbm.at[0, pl.ds(off, B)], sidx_v)
        pltpu.sync_copy(edge_hbm.at[1, pl.ds(off, B)], didx_v)
        pltpu.async_copy(h_hbm.at[sidx_v], rows_v, sem).wait()
        pltpu.sync_copy(rows_v, acc_sh.at[didx_v], add=True)
        return carry

    lax.fori_loop(0, nb, body, None)
    plsc.subcore_barrier()

    pltpu.sync_copy(acc_sh.at[pl.ds(r0, ROWS_PER_TILE)],
                    out_hbm.at[c, pl.ds(r0, ROWS_PER_TILE)])


_R = 1280


def _norm_cols(degp_ref):
    ns = lax.rsqrt(degp_ref[0, 0] + degp_ref[1, 0] + 1.0)
    nd = lax.rsqrt(degp_ref[0, 1] + degp_ref[1, 1] + 1.0)
    return ns, nd


def _prescale_body(feat_ref, degp_ref, h_ref):
    ns, _ = _norm_cols(degp_ref)
    h_ref[...] = feat_ref[...] * ns


def _layer1_body(aggp_ref, h_ref, degp_ref, w_ref, b_ref, out_ref):
    ns, nd = _norm_cols(degp_ref)
    agg = (aggp_ref[0] + aggp_ref[1] + h_ref[...]) * nd
    u = jnp.dot(agg, w_ref[...], preferred_element_type=jnp.float32,
                precision=lax.Precision.HIGHEST)
    u = jnp.maximum(u + b_ref[...], 0.0)
    out_ref[...] = u * ns


def _layer2_body(aggp_ref, h_ref, degp_ref, w_ref, b_ref, out_ref):
    _, nd = _norm_cols(degp_ref)
    agg = (aggp_ref[0] + aggp_ref[1] + h_ref[...]) * nd
    u = jnp.dot(agg, w_ref[...], preferred_element_type=jnp.float32,
                precision=lax.Precision.HIGHEST)
    out_ref[...] = u + b_ref[...]


_row_spec = pl.BlockSpec((_R, D), lambda i: (i, 0))
_degp_spec = pl.BlockSpec((NC, 2, _R, 1), lambda i: (0, 0, i, 0))
_aggp_spec = pl.BlockSpec((NC, _R, D), lambda i: (0, i, 0))
_w_spec = pl.BlockSpec((D, D), lambda i: (0, 0))
_b_spec = pl.BlockSpec((1, D), lambda i: (0, 0))

_prescale_call = pl.pallas_call(
    _prescale_body,
    grid=(NP // _R,),
    in_specs=[_row_spec, _degp_spec],
    out_specs=_row_spec,
    out_shape=jax.ShapeDtypeStruct((NP, D), jnp.float32),
)

_layer1_call = pl.pallas_call(
    _layer1_body,
    grid=(NP // _R,),
    in_specs=[_aggp_spec, _row_spec, _degp_spec, _w_spec, _b_spec],
    out_specs=_row_spec,
    out_shape=jax.ShapeDtypeStruct((NP, D), jnp.float32),
)

_layer2_call = pl.pallas_call(
    _layer2_body,
    grid=(NP // _R,),
    in_specs=[_aggp_spec, _row_spec, _degp_spec, _w_spec, _b_spec],
    out_specs=_row_spec,
    out_shape=jax.ShapeDtypeStruct((NP, D), jnp.float32),
)


@jax.jit
def kernel(features, edge_index, W1, b1, W2, b2):
    edge_index = edge_index.astype(jnp.int32)
    feat_p = jnp.zeros((NP, D), jnp.float32).at[:N].set(features)

    degp = _degree_kernel(edge_index)
    degp4 = degp.reshape(NC, 2, NP, 1)

    h = _prescale_call(feat_p, degp4)
    agg1 = _aggregate_kernel(h, edge_index)
    h1 = _layer1_call(agg1, h, degp4, W1, b1.reshape(1, D))
    agg2 = _aggregate_kernel(h1, edge_index)
    out = _layer2_call(agg2, h1, degp4, W2, b2.reshape(1, D))
    return out[:N]
